# baseline (speedup 1.0000x reference)
import functools
import jax
import jax.numpy as jnp
from jax import lax
from jax.experimental import pallas as pl
from jax.experimental.pallas import tpu as pltpu
from jax.experimental.pallas import tpu_sc as plsc

N = 10000
E = 320000
D = 128

NC = 2
NS = 16
NW = NC * NS
CHUNK = 128
G = 8
NCHUNKS = 2560
EPAD = NCHUNKS * CHUNK
NB = NCHUNKS // G
BITERS = NB // NW
ZCH = 80
NZ = N // ZCH
ZITERS = (NZ + NS - 1) // NS

ROW_BLOCK = 1000


def _spmm_body(h_hbm, src_hbm, dst_hbm, w_hbm, z_hbm, out_hbm,
               srcb, dstb, wb, rows0, rows1, acc_shared,
               sg0, sg1, ss0, ss1):
    cid = lax.axis_index("c")
    sid = lax.axis_index("s")
    wid = sid * NC + cid
    rows = (rows0, rows1)
    sg = (sg0, sg1)
    ss = (ss0, ss1)

    for i in range(ZITERS):
        c = sid + i * NS

        @pl.when(c < NZ)
        def _():
            pltpu.sync_copy(z_hbm, acc_shared.at[pl.ds(c * ZCH, ZCH)])

    plsc.subcore_barrier()

    def scale_chunk(rows_v, g):
        def scale(e, inner):
            wvec = plsc.load_gather(
                wb, [jnp.full((16,), g, jnp.int32),
                     jnp.full((16,), e, jnp.int32)])
            for j in range(D // 16):
                sl = pl.ds(j * 16, 16)
                rows_v[e, sl] = rows_v[e, sl] * wvec
            return inner

        lax.fori_loop(0, CHUNK, scale, 0, unroll=2)

    def block_iter(i, carry):
        b = wid + i * NW

        @pl.when(b < NB)
        def _():
            pltpu.sync_copy(src_hbm.at[pl.ds(b * G, G)], srcb)
            pltpu.sync_copy(dst_hbm.at[pl.ds(b * G, G)], dstb)
            pltpu.sync_copy(w_hbm.at[pl.ds(b * G, G)], wb)
            gdesc = [None] * G
            sdesc = [None] * G
            gdesc[0] = pltpu.async_copy(h_hbm.at[srcb.at[0]], rows[0], sg[0])
            for g in range(G):
                buf = g % 2
                if g + 1 < G:
                    nb_ = 1 - buf
                    if g >= 1:
                        sdesc[g - 1].wait()
                    gdesc[g + 1] = pltpu.async_copy(
                        h_hbm.at[srcb.at[g + 1]], rows[nb_], sg[nb_])
                gdesc[g].wait()
                scale_chunk(rows[buf], g)
                sdesc[g] = pltpu.async_copy(
                    rows[buf], acc_shared.at[dstb.at[g]], ss[buf], add=True)
            sdesc[G - 2].wait()
            sdesc[G - 1].wait()

        return carry

    lax.fori_loop(0, BITERS, block_iter, 0)
    plsc.subcore_barrier()

    for i in range(ZITERS):
        c = sid + i * NS

        @pl.when(c < NZ)
        def _():
            pltpu.sync_copy(acc_shared.at[pl.ds(c * ZCH, ZCH)],
                            out_hbm.at[cid, pl.ds(c * ZCH, ZCH)])


@jax.jit
def _spmm(h, src, dst, w, zeros):
    mesh = plsc.VectorSubcoreMesh(core_axis_name="c", subcore_axis_name="s")
    f = pl.kernel(
        _spmm_body,
        out_type=jax.ShapeDtypeStruct((NC, N, D), jnp.float32),
        mesh=mesh,
        compiler_params=pltpu.CompilerParams(needs_layout_passes=False),
        scratch_types=[
            pltpu.VMEM((G, CHUNK), jnp.int32),
            pltpu.VMEM((G, CHUNK), jnp.int32),
            pltpu.VMEM((G, CHUNK), jnp.float32),
            pltpu.VMEM((CHUNK, D), jnp.float32),
            pltpu.VMEM((CHUNK, D), jnp.float32),
            pltpu.VMEM_SHARED((N, D), jnp.float32),
            pltpu.SemaphoreType.DMA,
            pltpu.SemaphoreType.DMA,
            pltpu.SemaphoreType.DMA,
            pltpu.SemaphoreType.DMA,
        ],
    )
    return f(h, src, dst, w, zeros)


def _fc1_body(x_ref, w_ref, b_ref, o_ref):
    o_ref[...] = (
        jnp.dot(x_ref[...], w_ref[...], preferred_element_type=jnp.float32)
        + b_ref[...]
    )


@jax.jit
def _fc1(x, w, b):
    return pl.pallas_call(
        _fc1_body,
        grid=(N // ROW_BLOCK,),
        in_specs=[
            pl.BlockSpec((ROW_BLOCK, D), lambda i: (i, 0)),
            pl.BlockSpec((D, D), lambda i: (0, 0)),
            pl.BlockSpec((1, D), lambda i: (0, 0)),
        ],
        out_specs=pl.BlockSpec((ROW_BLOCK, D), lambda i: (i, 0)),
        out_shape=jax.ShapeDtypeStruct((N, D), jnp.float32),
    )(x, w, b)


def _fc2_body(p_ref, w_ref, b_ref, o_ref):
    h = jnp.maximum(p_ref[0] + p_ref[1], 0.0)
    y = jnp.dot(h, w_ref[...], preferred_element_type=jnp.float32) + b_ref[...]
    m = jnp.max(y, axis=1, keepdims=True)
    s = y - m
    o_ref[...] = s - jnp.log(jnp.sum(jnp.exp(s), axis=1, keepdims=True))


@jax.jit
def _fc2(parts, w, b):
    return pl.pallas_call(
        _fc2_body,
        grid=(N // ROW_BLOCK,),
        in_specs=[
            pl.BlockSpec((NC, ROW_BLOCK, D), lambda i: (0, i, 0)),
            pl.BlockSpec((D, D), lambda i: (0, 0)),
            pl.BlockSpec((1, D), lambda i: (0, 0)),
        ],
        out_specs=pl.BlockSpec((ROW_BLOCK, D), lambda i: (i, 0)),
        out_shape=jax.ShapeDtypeStruct((N, D), jnp.float32),
    )(parts, w, b)


def kernel(features, edge_index, edge_weight, W1, b1, W2, b2):
    pad = EPAD - E
    src = jnp.pad(edge_index[0].astype(jnp.int32), (0, pad)).reshape(
        NCHUNKS, CHUNK)
    dst = jnp.pad(edge_index[1].astype(jnp.int32), (0, pad)).reshape(
        NCHUNKS, CHUNK)
    w = jnp.pad(edge_weight, (0, pad)).reshape(NCHUNKS, CHUNK)
    h = _fc1(features, W1, b1.reshape(1, D))
    zeros = jnp.zeros((ZCH, D), jnp.float32)
    parts = _spmm(h, src, dst, w, zeros)
    return _fc2(parts, W2, b2.reshape(1, D))
